# Initial kernel scaffold; baseline (speedup 1.0000x reference)
import jax
import jax.numpy as jnp
from jax.experimental import pallas as pl


def kernel(classifications, regressions, anchors):
    raise NotImplementedError("write your pallas kernel here")


# speedup vs baseline: 11.5831x; 11.5831x over previous
import jax
import jax.numpy as jnp
from jax import lax
from jax.experimental import pallas as pl

N = 5000
C = 20
K = 200
PROB_THR = 0.05
NMS_THR = 0.45
VAR0, VAR1 = 0.1, 0.2
NEG = -1e9


def _body(cls_ref, reg_ref, anc_ref, out_ref):
    c = cls_ref[0]
    a = anc_ref[...]
    r = reg_ref[0]

    cmax = jnp.max(c, axis=0, keepdims=True)
    e = jnp.exp(c - cmax)
    probs = e / jnp.sum(e, axis=0, keepdims=True)

    acx, acy, aw, ah = a[0:1], a[1:2], a[2:3], a[3:4]
    rx, ry, rw, rh = r[0:1], r[1:2], r[2:3], r[3:4]
    cx = acx + rx * VAR0 * aw
    cy = acy + ry * VAR0 * ah
    w = aw * jnp.exp(rw * VAR1)
    h = ah * jnp.exp(rh * VAR1)
    X1 = jnp.clip(cx - w / 2.0, 0.0, 1.0)
    Y1 = jnp.clip(cy - h / 2.0, 0.0, 1.0)
    X2 = jnp.clip(cx + w / 2.0, 0.0, 1.0)
    Y2 = jnp.clip(cy + h / 2.0, 0.0, 1.0)
    A2 = (X2 - X1) * (Y2 - Y1)

    iota_n = lax.broadcasted_iota(jnp.int32, (1, N), 1)
    col_k = lax.broadcasted_iota(jnp.int32, (1, K), 1)

    s0 = jnp.where(probs[1:] > PROB_THR, probs[1:], NEG)
    z_ck = jnp.zeros((C, K), jnp.float32)

    def nms_step(t, carry):
        s, kx1, ky1, kx2, ky2, ksc = carry
        m = jnp.max(s, axis=1, keepdims=True)
        valid = m > (NEG / 2.0)
        idx = jnp.min(jnp.where(s == m, iota_n, N), axis=1, keepdims=True)
        onehot = iota_n == idx
        bx1 = jnp.sum(jnp.where(onehot, X1, 0.0), axis=1, keepdims=True)
        by1 = jnp.sum(jnp.where(onehot, Y1, 0.0), axis=1, keepdims=True)
        bx2 = jnp.sum(jnp.where(onehot, X2, 0.0), axis=1, keepdims=True)
        by2 = jnp.sum(jnp.where(onehot, Y2, 0.0), axis=1, keepdims=True)
        iw = jnp.maximum(jnp.minimum(bx2, X2) - jnp.maximum(bx1, X1), 0.0)
        ih = jnp.maximum(jnp.minimum(by2, Y2) - jnp.maximum(by1, Y1), 0.0)
        inter = iw * ih
        a1 = (bx2 - bx1) * (by2 - by1)
        iou = inter / (a1 + A2 - inter + 1e-12)
        supp = (iou > NMS_THR) | onehot
        s = jnp.where(valid & supp, NEG, s)
        hit = (col_k == t) & valid
        ksc = jnp.where(hit, m, ksc)
        kx1 = jnp.where(hit, bx1, kx1)
        ky1 = jnp.where(hit, by1, ky1)
        kx2 = jnp.where(hit, bx2, kx2)
        ky2 = jnp.where(hit, by2, ky2)
        return s, kx1, ky1, kx2, ky2, ksc

    _, kx1, ky1, kx2, ky2, ksc = lax.fori_loop(
        0, K, nms_step, (s0, z_ck, z_ck, z_ck, z_ck, z_ck))

    row_c = lax.broadcasted_iota(jnp.int32, (C, 1), 0)
    flat = row_c * K + col_k
    z_1k = jnp.zeros((1, K), jnp.float32)

    def top_step(t, carry):
        S, ox1, oy1, ox2, oy2, osc, olb = carry
        g = jnp.max(S)
        pick = g > 0.0
        fi = jnp.min(jnp.where(S == g, flat, C * K))
        oh = flat == fi
        gx1 = jnp.sum(jnp.where(oh, kx1, 0.0))
        gy1 = jnp.sum(jnp.where(oh, ky1, 0.0))
        gx2 = jnp.sum(jnp.where(oh, kx2, 0.0))
        gy2 = jnp.sum(jnp.where(oh, ky2, 0.0))
        lab = (fi // K + 1).astype(jnp.float32)
        hit = col_k == t
        ox1 = jnp.where(hit, jnp.where(pick, gx1, 0.0), ox1)
        oy1 = jnp.where(hit, jnp.where(pick, gy1, 0.0), oy1)
        ox2 = jnp.where(hit, jnp.where(pick, gx2, 0.0), ox2)
        oy2 = jnp.where(hit, jnp.where(pick, gy2, 0.0), oy2)
        osc = jnp.where(hit, jnp.where(pick, g, 0.0), osc)
        olb = jnp.where(hit, jnp.where(pick, lab, 0.0), olb)
        S = jnp.where(pick & oh, jnp.float32(-1.0), S)
        return S, ox1, oy1, ox2, oy2, osc, olb

    _, ox1, oy1, ox2, oy2, osc, olb = lax.fori_loop(
        0, K, top_step, (ksc, z_1k, z_1k, z_1k, z_1k, z_1k, z_1k))

    out_ref[0, 0:1, :] = ox1
    out_ref[0, 1:2, :] = oy1
    out_ref[0, 2:3, :] = ox2
    out_ref[0, 3:4, :] = oy2
    out_ref[0, 4:5, :] = osc
    out_ref[0, 5:6, :] = olb


@jax.jit
def kernel(classifications, regressions, anchors):
    B = classifications.shape[0]
    cls_t = jnp.transpose(classifications, (0, 2, 1))
    reg_t = jnp.transpose(regressions, (0, 2, 1))
    anc_t = anchors.T
    out = pl.pallas_call(
        _body,
        grid=(B,),
        in_specs=[
            pl.BlockSpec((1, 21, N), lambda i: (i, 0, 0)),
            pl.BlockSpec((1, 4, N), lambda i: (i, 0, 0)),
            pl.BlockSpec((4, N), lambda i: (0, 0)),
        ],
        out_specs=pl.BlockSpec((1, 6, K), lambda i: (i, 0, 0)),
        out_shape=jax.ShapeDtypeStruct((B, 6, K), jnp.float32),
    )(cls_t, reg_t, anc_t)
    return jnp.transpose(out, (0, 2, 1))


# speedup vs baseline: 15.1522x; 1.3081x over previous
import jax
import jax.numpy as jnp
from jax import lax
from jax.experimental import pallas as pl

N = 5000
C = 20
K = 200
PROB_THR = 0.05
NMS_THR = 0.45
VAR0, VAR1 = 0.1, 0.2
NEG = -1e9


def _softmax21(c):
    cmax = jnp.max(c, axis=0, keepdims=True)
    e = jnp.exp(c - cmax)
    return e / jnp.sum(e, axis=0, keepdims=True)


def _decode(r, a):
    acx, acy, aw, ah = a[0:1], a[1:2], a[2:3], a[3:4]
    rx, ry, rw, rh = r[0:1], r[1:2], r[2:3], r[3:4]
    cx = acx + rx * VAR0 * aw
    cy = acy + ry * VAR0 * ah
    w = aw * jnp.exp(rw * VAR1)
    h = ah * jnp.exp(rh * VAR1)
    x1 = jnp.clip(cx - w / 2.0, 0.0, 1.0)
    y1 = jnp.clip(cy - h / 2.0, 0.0, 1.0)
    x2 = jnp.clip(cx + w / 2.0, 0.0, 1.0)
    y2 = jnp.clip(cy + h / 2.0, 0.0, 1.0)
    return x1, y1, x2, y2


def _body(cls_ref, reg_ref, anc_ref, out_ref):
    c = cls_ref[...]
    rg = reg_ref[...]
    a = anc_ref[...]

    p0 = _softmax21(c[0:21])
    p1 = _softmax21(c[21:42])

    b0 = _decode(rg[0:4], a)
    b1 = _decode(rg[4:8], a)
    X1, Y1, X2, Y2 = (
        jnp.concatenate(
            [jnp.broadcast_to(u0, (C, N)), jnp.broadcast_to(u1, (C, N))], axis=0)
        for u0, u1 in zip(b0, b1))
    A2 = (X2 - X1) * (Y2 - Y1)

    iota_n = lax.broadcasted_iota(jnp.int32, (1, N), 1)
    col_k = lax.broadcasted_iota(jnp.int32, (1, K), 1)

    s0 = jnp.concatenate([p0[1:], p1[1:]], axis=0)
    s0 = jnp.where(s0 > PROB_THR, s0, NEG)
    z_ck = jnp.zeros((2 * C, K), jnp.float32)

    def nms_step(t, carry):
        s, kx1, ky1, kx2, ky2, ksc = carry
        m = jnp.max(s, axis=1, keepdims=True)
        valid = m > (NEG / 2.0)
        idx = jnp.min(jnp.where(s == m, iota_n, N), axis=1, keepdims=True)
        onehot = iota_n == idx
        bx1 = jnp.sum(jnp.where(onehot, X1, 0.0), axis=1, keepdims=True)
        by1 = jnp.sum(jnp.where(onehot, Y1, 0.0), axis=1, keepdims=True)
        bx2 = jnp.sum(jnp.where(onehot, X2, 0.0), axis=1, keepdims=True)
        by2 = jnp.sum(jnp.where(onehot, Y2, 0.0), axis=1, keepdims=True)
        iw = jnp.maximum(jnp.minimum(bx2, X2) - jnp.maximum(bx1, X1), 0.0)
        ih = jnp.maximum(jnp.minimum(by2, Y2) - jnp.maximum(by1, Y1), 0.0)
        inter = iw * ih
        a1 = (bx2 - bx1) * (by2 - by1)
        iou = inter / (a1 + A2 - inter + 1e-12)
        supp = (iou > NMS_THR) | onehot
        s = jnp.where(valid & supp, NEG, s)
        hit = (col_k == t) & valid
        ksc = jnp.where(hit, m, ksc)
        kx1 = jnp.where(hit, bx1, kx1)
        ky1 = jnp.where(hit, by1, ky1)
        kx2 = jnp.where(hit, bx2, kx2)
        ky2 = jnp.where(hit, by2, ky2)
        return s, kx1, ky1, kx2, ky2, ksc

    _, kx1, ky1, kx2, ky2, ksc = lax.fori_loop(
        0, K, nms_step, (s0, z_ck, z_ck, z_ck, z_ck, z_ck))

    row_c = lax.broadcasted_iota(jnp.int32, (C, 1), 0)
    flat = row_c * K + col_k
    z_1k = jnp.zeros((1, K), jnp.float32)

    halves = []
    for h in (0, 1):
        halves.append(tuple(arr[h * C:(h + 1) * C]
                            for arr in (ksc, kx1, ky1, kx2, ky2)))

    def top_step(t, carry):
        S0, S1, outs = carry
        hit = col_k == t
        new_S = []
        new_outs = []
        for h, S in ((0, S0), (1, S1)):
            _, hx1, hy1, hx2, hy2 = halves[h]
            g = jnp.max(S)
            pick = g > 0.0
            fi = jnp.min(jnp.where(S == g, flat, C * K))
            oh = flat == fi
            gx1 = jnp.sum(jnp.where(oh, hx1, 0.0))
            gy1 = jnp.sum(jnp.where(oh, hy1, 0.0))
            gx2 = jnp.sum(jnp.where(oh, hx2, 0.0))
            gy2 = jnp.sum(jnp.where(oh, hy2, 0.0))
            lab = (fi // K + 1).astype(jnp.float32)
            vals = (gx1, gy1, gx2, gy2, g, lab)
            base = h * 6
            for j in range(6):
                new_outs.append(jnp.where(
                    hit, jnp.where(pick, vals[j], 0.0), outs[base + j]))
            new_S.append(jnp.where(pick & oh, jnp.float32(-1.0), S))
        return new_S[0], new_S[1], tuple(new_outs)

    _, _, outs = lax.fori_loop(
        0, K, top_step, (halves[0][0], halves[1][0], (z_1k,) * 12))

    for j in range(12):
        out_ref[j:j + 1, :] = outs[j]


@jax.jit
def kernel(classifications, regressions, anchors):
    B = classifications.shape[0]
    cls_t = jnp.transpose(classifications, (0, 2, 1)).reshape(B * 21, N)
    reg_t = jnp.transpose(regressions, (0, 2, 1)).reshape(B * 4, N)
    anc_t = anchors.T
    out = pl.pallas_call(
        _body,
        grid=(1,),
        in_specs=[
            pl.BlockSpec((B * 21, N), lambda i: (0, 0)),
            pl.BlockSpec((B * 4, N), lambda i: (0, 0)),
            pl.BlockSpec((4, N), lambda i: (0, 0)),
        ],
        out_specs=pl.BlockSpec((6 * B, K), lambda i: (0, 0)),
        out_shape=jax.ShapeDtypeStruct((6 * B, K), jnp.float32),
    )(cls_t, reg_t, anc_t)
    return jnp.transpose(out.reshape(B, 6, K), (0, 2, 1))
